# baseline (speedup 1.0000x reference)
import functools

import jax
import jax.numpy as jnp
from jax import lax
from jax.experimental import pallas as pl
from jax.experimental.pallas import tpu as pltpu
from jax.experimental.pallas import tpu_sc as plsc

B = 16384
F = 26
V = 1000
D = 64
NUM = 13
H = 1024
FD = F * D
BF = B * F
NUMP = 128
DW = D // 2

NC, NS = 2, 16
NW = NC * NS
ROWS_W = BF // NW
CHUNK = 128
CH = ROWS_W // CHUNK

_sc_mesh = plsc.VectorSubcoreMesh(core_axis_name="c", subcore_axis_name="s")


@functools.partial(
    pl.kernel,
    out_type=jax.ShapeDtypeStruct((BF, D), jnp.float32),
    mesh=_sc_mesh,
    scratch_types=[
        pltpu.VMEM((CH, CHUNK), jnp.int32),
        pltpu.VMEM((2, CHUNK, D), jnp.float32),
        pltpu.SemaphoreType.DMA,
        pltpu.SemaphoreType.DMA,
    ],
    compiler_params=pltpu.CompilerParams(use_tc_tiling_on_sc=False),
)
def _sc_gather(table_hbm, idx_hbm, out_hbm, idx_v, rows_v, gsem, osem):
    wid = lax.axis_index("s") * NC + lax.axis_index("c")
    pltpu.sync_copy(idx_hbm.at[pl.ds(wid * CH, CH)], idx_v)
    base_row = wid * ROWS_W

    def out_slice(j):
        return out_hbm.at[pl.ds(base_row + j * CHUNK, CHUNK)]

    def body(j, carry):
        pltpu.async_copy(table_hbm.at[idx_v.at[j]], rows_v.at[0], gsem).wait()
        pltpu.sync_copy(rows_v.at[0], out_slice(j))
        return carry

    lax.fori_loop(0, CH, body, 0)


def _mlp_body(emb_ref, num_ref, mean_ref, std_ref, w1e_ref, w1n_ref,
              b1_ref, w2_ref, b2_ref, out_ref):
    num = (num_ref[...] - mean_ref[...]) / std_ref[...]
    acc = jnp.dot(emb_ref[...], w1e_ref[...], preferred_element_type=jnp.float32)
    acc = acc + jnp.dot(num, w1n_ref[...], preferred_element_type=jnp.float32)
    x = jnp.maximum(acc + b1_ref[...], 0.0)
    out_ref[...] = jnp.sum(x * w2_ref[...], axis=1, keepdims=True) + b2_ref[...]


BB = 512


def _mlp(emb, num_p, mean_p, std_p, w1e, w1n, b1r, w2r, b2r):
    grid = (B // BB,)
    return pl.pallas_call(
        _mlp_body,
        grid=grid,
        in_specs=[
            pl.BlockSpec((BB, FD), lambda i: (i, 0)),
            pl.BlockSpec((BB, NUMP), lambda i: (i, 0)),
            pl.BlockSpec((1, NUMP), lambda i: (0, 0)),
            pl.BlockSpec((1, NUMP), lambda i: (0, 0)),
            pl.BlockSpec((FD, H), lambda i: (0, 0)),
            pl.BlockSpec((NUMP, H), lambda i: (0, 0)),
            pl.BlockSpec((1, H), lambda i: (0, 0)),
            pl.BlockSpec((1, H), lambda i: (0, 0)),
            pl.BlockSpec((1, 1), lambda i: (0, 0)),
        ],
        out_specs=pl.BlockSpec((BB, 1), lambda i: (i, 0)),
        out_shape=jax.ShapeDtypeStruct((B, 1), jnp.float32),
    )(emb, num_p, mean_p, std_p, w1e, w1n, b1r, w2r, b2r)


def kernel(cat_indices, numericals, emb_tables, norm_mean, norm_std, W1, b1, W2, b2):
    tab = emb_tables.reshape(F * V, D)
    offs = (jnp.arange(F, dtype=jnp.int32) * V)[None, :]
    flat_idx = (cat_indices.astype(jnp.int32) + offs).reshape(NW * CH, CHUNK)
    emb = _sc_gather(tab, flat_idx).reshape(B, FD).astype(jnp.bfloat16)

    num_p = jnp.pad(numericals, ((0, 0), (0, NUMP - NUM)))
    mean_p = jnp.pad(norm_mean, (0, NUMP - NUM)).reshape(1, NUMP)
    std_p = jnp.pad(norm_std, (0, NUMP - NUM), constant_values=1.0).reshape(1, NUMP)
    w1e = W1[:FD].astype(jnp.bfloat16)
    w1n = jnp.pad(W1[FD:], ((0, NUMP - NUM), (0, 0)))
    return _mlp(emb, num_p, mean_p, std_p, w1e, w1n,
                b1.reshape(1, H), W2.reshape(1, H), b2.reshape(1, 1))


# speedup vs baseline: 1.4313x; 1.4313x over previous
import functools

import jax
import jax.numpy as jnp
from jax import lax
from jax.experimental import pallas as pl
from jax.experimental.pallas import tpu as pltpu
from jax.experimental.pallas import tpu_sc as plsc

B = 16384
F = 26
V = 1000
D = 64
NUM = 13
H = 1024
FD = F * D
BF = B * F
NUMP = 128
DW = D // 2

NC, NS = 2, 16
NW = NC * NS
ROWS_W = BF // NW
CHUNK = 128
CH = ROWS_W // CHUNK

_sc_mesh = plsc.VectorSubcoreMesh(core_axis_name="c", subcore_axis_name="s")


@functools.partial(
    pl.kernel,
    out_type=jax.ShapeDtypeStruct((BF, D), jnp.float32),
    mesh=_sc_mesh,
    scratch_types=[
        pltpu.VMEM((CH, CHUNK), jnp.int32),
        pltpu.VMEM((2, CHUNK, D), jnp.float32),
        pltpu.SemaphoreType.DMA,
        pltpu.SemaphoreType.DMA,
    ],
    compiler_params=pltpu.CompilerParams(use_tc_tiling_on_sc=False),
)
def _sc_gather(table_hbm, idx_hbm, out_hbm, idx_v, rows_v, gsem, osem):
    wid = lax.axis_index("s") * NC + lax.axis_index("c")
    pltpu.sync_copy(idx_hbm.at[pl.ds(wid * CH, CH)], idx_v)
    base_row = wid * ROWS_W

    def out_slice(j):
        return out_hbm.at[pl.ds(base_row + j * CHUNK, CHUNK)]

    def body(j, carry):
        pltpu.async_copy(table_hbm.at[idx_v.at[j]], rows_v.at[0], gsem).wait()
        pltpu.sync_copy(rows_v.at[0], out_slice(j))
        return carry

    lax.fori_loop(0, CH, body, 0)


def _mlp_body(emb_ref, num_ref, mean_ref, std_ref, w1e_ref, w1n_ref,
              b1_ref, w2_ref, b2_ref, out_ref):
    num = (num_ref[...] - mean_ref[...]) / std_ref[...]
    acc = jnp.dot(emb_ref[...].astype(jnp.bfloat16), w1e_ref[...],
                  preferred_element_type=jnp.float32)
    acc = acc + jnp.dot(num, w1n_ref[...], preferred_element_type=jnp.float32)
    x = jnp.maximum(acc + b1_ref[...], 0.0)
    out_ref[...] = jnp.sum(x * w2_ref[...], axis=1, keepdims=True) + b2_ref[...]


BB = 512


def _mlp(emb, num_p, mean_p, std_p, w1e, w1n, b1r, w2r, b2r):
    grid = (B // BB,)
    return pl.pallas_call(
        _mlp_body,
        grid=grid,
        in_specs=[
            pl.BlockSpec((BB, FD), lambda i: (i, 0)),
            pl.BlockSpec((BB, NUMP), lambda i: (i, 0)),
            pl.BlockSpec((1, NUMP), lambda i: (0, 0)),
            pl.BlockSpec((1, NUMP), lambda i: (0, 0)),
            pl.BlockSpec((FD, H), lambda i: (0, 0)),
            pl.BlockSpec((NUMP, H), lambda i: (0, 0)),
            pl.BlockSpec((1, H), lambda i: (0, 0)),
            pl.BlockSpec((1, H), lambda i: (0, 0)),
            pl.BlockSpec((1, 1), lambda i: (0, 0)),
        ],
        out_specs=pl.BlockSpec((BB, 1), lambda i: (i, 0)),
        out_shape=jax.ShapeDtypeStruct((B, 1), jnp.float32),
    )(emb, num_p, mean_p, std_p, w1e, w1n, b1r, w2r, b2r)


def kernel(cat_indices, numericals, emb_tables, norm_mean, norm_std, W1, b1, W2, b2):
    tab = emb_tables.reshape(F * V, D)
    offs = (jnp.arange(F, dtype=jnp.int32) * V)[None, :]
    flat_idx = (cat_indices.astype(jnp.int32) + offs).reshape(NW * CH, CHUNK)
    emb = _sc_gather(tab, flat_idx).reshape(B, FD)

    num_p = jnp.pad(numericals, ((0, 0), (0, NUMP - NUM)))
    mean_p = jnp.pad(norm_mean, (0, NUMP - NUM)).reshape(1, NUMP)
    std_p = jnp.pad(norm_std, (0, NUMP - NUM), constant_values=1.0).reshape(1, NUMP)
    w1e = W1[:FD].astype(jnp.bfloat16)
    w1n = jnp.pad(W1[FD:], ((0, NUMP - NUM), (0, 0)))
    return _mlp(emb, num_p, mean_p, std_p, w1e, w1n,
                b1.reshape(1, H), W2.reshape(1, H), b2.reshape(1, 1))


# speedup vs baseline: 1.7015x; 1.1888x over previous
import functools

import jax
import jax.numpy as jnp
from jax import lax
from jax.experimental import pallas as pl
from jax.experimental.pallas import tpu as pltpu
from jax.experimental.pallas import tpu_sc as plsc

B = 16384
F = 26
V = 1000
D = 64
NUM = 13
H = 1024
FD = F * D
CT = FD // 128
BF = B * F
NUMP = 128

NC, NS = 2, 16
NW = NC * NS
CHUNK = 128
NCHUNKS = BF // CHUNK
CH = NCHUNKS // NW

_sc_mesh = plsc.VectorSubcoreMesh(core_axis_name="c", subcore_axis_name="s")


@functools.partial(
    pl.kernel,
    out_type=jax.ShapeDtypeStruct((BF, D), jnp.float32),
    mesh=_sc_mesh,
    scratch_types=[
        pltpu.VMEM((CH, CHUNK), jnp.int32),
        pltpu.VMEM((2, CHUNK, D), jnp.float32),
        pltpu.SemaphoreType.DMA,
        pltpu.SemaphoreType.DMA,
    ],
    compiler_params=pltpu.CompilerParams(use_tc_tiling_on_sc=False),
)
def _sc_gather(table_hbm, idx_hbm, out_hbm, idx_v, rows_v, gsem, osem):
    wid = lax.axis_index("s") * NC + lax.axis_index("c")
    pltpu.sync_copy(idx_hbm.at[pl.ds(wid * CH, CH)], idx_v)
    base_row = wid * CH * CHUNK

    def out_slice(j):
        return out_hbm.at[pl.ds(base_row + j * CHUNK, CHUNK)]

    def body(j, carry):
        pltpu.async_copy(table_hbm.at[idx_v.at[j]], rows_v.at[0], gsem).wait()
        pltpu.sync_copy(rows_v.at[0], out_slice(j))
        return carry

    lax.fori_loop(0, CH, body, 0)


def _mlp_body(emb_ref, num_ref, mean_ref, std_ref, w1e_ref, w1n_ref,
              b1_ref, w2_ref, b2_ref, out_ref):
    num = (num_ref[...] - mean_ref[...]) / std_ref[...]
    acc = jnp.dot(num, w1n_ref[...], preferred_element_type=jnp.float32)
    for c in range(CT):
        blk = emb_ref[c].astype(jnp.bfloat16)
        acc = acc + jnp.dot(blk, w1e_ref[c], preferred_element_type=jnp.float32)
    x = jnp.maximum(acc + b1_ref[...], 0.0)
    out_ref[...] = jnp.sum(x * w2_ref[...], axis=1, keepdims=True) + b2_ref[...]


BB = 512


def _mlp(emb3, num_p, mean_p, std_p, w1e, w1n, b1r, w2r, b2r):
    grid = (B // BB,)
    return pl.pallas_call(
        _mlp_body,
        grid=grid,
        in_specs=[
            pl.BlockSpec((CT, BB, 128), lambda i: (0, i, 0)),
            pl.BlockSpec((BB, NUMP), lambda i: (i, 0)),
            pl.BlockSpec((1, NUMP), lambda i: (0, 0)),
            pl.BlockSpec((1, NUMP), lambda i: (0, 0)),
            pl.BlockSpec((CT, 128, H), lambda i: (0, 0, 0)),
            pl.BlockSpec((NUMP, H), lambda i: (0, 0)),
            pl.BlockSpec((1, H), lambda i: (0, 0)),
            pl.BlockSpec((1, H), lambda i: (0, 0)),
            pl.BlockSpec((1, 1), lambda i: (0, 0)),
        ],
        out_specs=pl.BlockSpec((BB, 1), lambda i: (i, 0)),
        out_shape=jax.ShapeDtypeStruct((B, 1), jnp.float32),
    )(emb3, num_p, mean_p, std_p, w1e, w1n, b1r, w2r, b2r)


def kernel(cat_indices, numericals, emb_tables, norm_mean, norm_std, W1, b1, W2, b2):
    tab = emb_tables.reshape(F * V, D)
    offs = (jnp.arange(F, dtype=jnp.int32) * V)[None, :]
    fi = cat_indices.astype(jnp.int32) + offs
    fi = fi.reshape(B // 64, 64, CT, 2).transpose(2, 0, 1, 3).reshape(NCHUNKS, CHUNK)
    flat = _sc_gather(tab, fi)
    emb3 = flat.reshape(CT, B, 128)

    num_p = jnp.pad(numericals, ((0, 0), (0, NUMP - NUM)))
    mean_p = jnp.pad(norm_mean, (0, NUMP - NUM)).reshape(1, NUMP)
    std_p = jnp.pad(norm_std, (0, NUMP - NUM), constant_values=1.0).reshape(1, NUMP)
    w1e = W1[:FD].astype(jnp.bfloat16).reshape(CT, 128, H)
    w1n = jnp.pad(W1[FD:], ((0, NUMP - NUM), (0, 0)))
    return _mlp(emb3, num_p, mean_p, std_p, w1e, w1n,
                b1.reshape(1, H), W2.reshape(1, H), b2.reshape(1, 1))


# speedup vs baseline: 2.2065x; 1.2968x over previous
import functools

import jax
import jax.numpy as jnp
from jax import lax
from jax.experimental import pallas as pl
from jax.experimental.pallas import tpu as pltpu
from jax.experimental.pallas import tpu_sc as plsc

B = 16384
F = 26
V = 1000
D = 64
NUM = 13
H = 1024
FD = F * D
CT = FD // 128
BF = B * F
NUMP = 128

NC, NS = 2, 16
NW = NC * NS
CHUNK = 128
NCHUNKS = BF // CHUNK
CH = NCHUNKS // NW

_sc_mesh = plsc.VectorSubcoreMesh(core_axis_name="c", subcore_axis_name="s")


@functools.partial(
    pl.kernel,
    out_type=jax.ShapeDtypeStruct((BF, D), jnp.float32),
    mesh=_sc_mesh,
    scratch_types=[
        pltpu.VMEM((CH, CHUNK), jnp.int32),
        pltpu.VMEM((2, CHUNK, D), jnp.float32),
        pltpu.SemaphoreType.DMA,
        pltpu.SemaphoreType.DMA,
    ],
    compiler_params=pltpu.CompilerParams(use_tc_tiling_on_sc=False),
)
def _sc_gather(table_hbm, idx_hbm, out_hbm, idx_v, rows_v, gsem, osem):
    wid = lax.axis_index("s") * NC + lax.axis_index("c")
    pltpu.sync_copy(idx_hbm.at[pl.ds(wid * CH, CH)], idx_v)
    base_row = wid * CH * CHUNK

    def out_slice(j):
        return out_hbm.at[pl.ds(base_row + j * CHUNK, CHUNK)]

    def body(j, carry):
        buf = lax.rem(j, 2)
        @pl.when(j >= 2)
        def _():
            pltpu.make_async_copy(rows_v.at[buf], out_slice(j - 2), osem).wait()

        pltpu.async_copy(table_hbm.at[idx_v.at[j]], rows_v.at[buf], gsem).wait()
        pltpu.async_copy(rows_v.at[buf], out_slice(j), osem)
        return carry

    lax.fori_loop(0, CH, body, 0)
    pltpu.make_async_copy(rows_v.at[0], out_slice(CH - 2), osem).wait()
    pltpu.make_async_copy(rows_v.at[1], out_slice(CH - 1), osem).wait()


def _mlp_body(emb_ref, num_ref, mean_ref, std_ref, w1e_ref, w1n_ref,
              b1_ref, w2_ref, b2_ref, out_ref):
    num = (num_ref[...] - mean_ref[...]) / std_ref[...]
    acc = jnp.dot(num, w1n_ref[...], preferred_element_type=jnp.float32)
    emb = jnp.concatenate(
        [emb_ref[c].astype(jnp.bfloat16) for c in range(CT)], axis=1)
    acc = acc + jnp.dot(emb, w1e_ref[...], preferred_element_type=jnp.float32)
    x = jnp.maximum(acc + b1_ref[...], 0.0)
    out_ref[...] = jnp.sum(x * w2_ref[...], axis=1, keepdims=True) + b2_ref[...]


BB = 512


def _mlp(emb3, num_p, mean_p, std_p, w1e, w1n, b1r, w2r, b2r):
    grid = (B // BB,)
    return pl.pallas_call(
        _mlp_body,
        grid=grid,
        in_specs=[
            pl.BlockSpec((CT, BB, 128), lambda i: (0, i, 0)),
            pl.BlockSpec((BB, NUMP), lambda i: (i, 0)),
            pl.BlockSpec((1, NUMP), lambda i: (0, 0)),
            pl.BlockSpec((1, NUMP), lambda i: (0, 0)),
            pl.BlockSpec((FD, H), lambda i: (0, 0)),
            pl.BlockSpec((NUMP, H), lambda i: (0, 0)),
            pl.BlockSpec((1, H), lambda i: (0, 0)),
            pl.BlockSpec((1, H), lambda i: (0, 0)),
            pl.BlockSpec((1, 1), lambda i: (0, 0)),
        ],
        out_specs=pl.BlockSpec((BB, 1), lambda i: (i, 0)),
        out_shape=jax.ShapeDtypeStruct((B, 1), jnp.float32),
    )(emb3, num_p, mean_p, std_p, w1e, w1n, b1r, w2r, b2r)


def kernel(cat_indices, numericals, emb_tables, norm_mean, norm_std, W1, b1, W2, b2):
    tab = emb_tables.reshape(F * V, D)
    offs = (jnp.arange(F, dtype=jnp.int32) * V)[None, :]
    fi = cat_indices.astype(jnp.int32) + offs
    fi = fi.reshape(B // 64, 64, CT, 2).transpose(2, 0, 1, 3).reshape(NCHUNKS, CHUNK)
    flat = _sc_gather(tab, fi)
    emb3 = flat.reshape(CT, B, 128)

    num_p = jnp.pad(numericals, ((0, 0), (0, NUMP - NUM)))
    mean_p = jnp.pad(norm_mean, (0, NUMP - NUM)).reshape(1, NUMP)
    std_p = jnp.pad(norm_std, (0, NUMP - NUM), constant_values=1.0).reshape(1, NUMP)
    w1e = W1[:FD].astype(jnp.bfloat16)
    w1n = jnp.pad(W1[FD:], ((0, NUMP - NUM), (0, 0)))
    return _mlp(emb3, num_p, mean_p, std_p, w1e, w1n,
                b1.reshape(1, H), W2.reshape(1, H), b2.reshape(1, 1))


# speedup vs baseline: 2.4249x; 1.0990x over previous
import functools

import jax
import jax.numpy as jnp
from jax import lax
from jax.experimental import pallas as pl
from jax.experimental.pallas import tpu as pltpu
from jax.experimental.pallas import tpu_sc as plsc

B = 16384
F = 26
V = 1000
D = 64
NUM = 13
H = 1024
FD = F * D
CT = FD // 128
NUMP = 128

NSPLIT = 4
BS = B // NSPLIT
BFS = BS * F

NC, NS = 2, 16
NW = NC * NS
CHUNK = 128
NCHUNKS = BFS // CHUNK
CH = NCHUNKS // NW

_sc_mesh = plsc.VectorSubcoreMesh(core_axis_name="c", subcore_axis_name="s")


@functools.partial(
    pl.kernel,
    out_type=jax.ShapeDtypeStruct((BFS, D), jnp.float32),
    mesh=_sc_mesh,
    scratch_types=[
        pltpu.VMEM((CH, CHUNK), jnp.int32),
        pltpu.VMEM((2, CHUNK, D), jnp.float32),
        pltpu.SemaphoreType.DMA,
        pltpu.SemaphoreType.DMA,
    ],
    compiler_params=pltpu.CompilerParams(use_tc_tiling_on_sc=False),
)
def _sc_gather(table_hbm, idx_hbm, out_hbm, idx_v, rows_v, gsem, osem):
    wid = lax.axis_index("s") * NC + lax.axis_index("c")
    pltpu.sync_copy(idx_hbm.at[pl.ds(wid * CH, CH)], idx_v)
    base_row = wid * CH * CHUNK

    def out_slice(j):
        return out_hbm.at[pl.ds(base_row + j * CHUNK, CHUNK)]

    def body(j, carry):
        buf = lax.rem(j, 2)
        @pl.when(j >= 2)
        def _():
            pltpu.make_async_copy(rows_v.at[buf], out_slice(j - 2), osem).wait()

        pltpu.async_copy(table_hbm.at[idx_v.at[j]], rows_v.at[buf], gsem).wait()
        pltpu.async_copy(rows_v.at[buf], out_slice(j), osem)
        return carry

    lax.fori_loop(0, CH, body, 0)
    pltpu.make_async_copy(rows_v.at[0], out_slice(CH - 2), osem).wait()
    pltpu.make_async_copy(rows_v.at[1], out_slice(CH - 1), osem).wait()


def _mlp_body(emb_ref, num_ref, mean_ref, std_ref, w1e_ref, w1n_ref,
              b1_ref, w2_ref, b2_ref, out_ref):
    num = (num_ref[...] - mean_ref[...]) / std_ref[...]
    acc = jnp.dot(num, w1n_ref[...], preferred_element_type=jnp.float32)
    emb = jnp.concatenate(
        [emb_ref[c].astype(jnp.bfloat16) for c in range(CT)], axis=1)
    acc = acc + jnp.dot(emb, w1e_ref[...], preferred_element_type=jnp.float32)
    x = jnp.maximum(acc + b1_ref[...], 0.0)
    out_ref[...] = jnp.sum(x * w2_ref[...], axis=1, keepdims=True) + b2_ref[...]


BB = 512


def _mlp(emb3, num_p, mean_p, std_p, w1e, w1n, b1r, w2r, b2r):
    grid = (BS // BB,)
    return pl.pallas_call(
        _mlp_body,
        grid=grid,
        in_specs=[
            pl.BlockSpec((CT, BB, 128), lambda i: (0, i, 0)),
            pl.BlockSpec((BB, NUMP), lambda i: (i, 0)),
            pl.BlockSpec((1, NUMP), lambda i: (0, 0)),
            pl.BlockSpec((1, NUMP), lambda i: (0, 0)),
            pl.BlockSpec((FD, H), lambda i: (0, 0)),
            pl.BlockSpec((NUMP, H), lambda i: (0, 0)),
            pl.BlockSpec((1, H), lambda i: (0, 0)),
            pl.BlockSpec((1, H), lambda i: (0, 0)),
            pl.BlockSpec((1, 1), lambda i: (0, 0)),
        ],
        out_specs=pl.BlockSpec((BB, 1), lambda i: (i, 0)),
        out_shape=jax.ShapeDtypeStruct((BS, 1), jnp.float32),
    )(emb3, num_p, mean_p, std_p, w1e, w1n, b1r, w2r, b2r)


def kernel(cat_indices, numericals, emb_tables, norm_mean, norm_std, W1, b1, W2, b2):
    tab = emb_tables.reshape(F * V, D)
    offs = (jnp.arange(F, dtype=jnp.int32) * V)[None, :]
    fi = cat_indices.astype(jnp.int32) + offs
    fi = fi.reshape(NSPLIT, BS // 64, 64, CT, 2).transpose(0, 3, 1, 2, 4)
    fi = fi.reshape(NSPLIT, NCHUNKS, CHUNK)

    num_p = jnp.pad(numericals, ((0, 0), (0, NUMP - NUM)))
    mean_p = jnp.pad(norm_mean, (0, NUMP - NUM)).reshape(1, NUMP)
    std_p = jnp.pad(norm_std, (0, NUMP - NUM), constant_values=1.0).reshape(1, NUMP)
    w1e = W1[:FD].astype(jnp.bfloat16)
    w1n = jnp.pad(W1[FD:], ((0, NUMP - NUM), (0, 0)))
    b1r, w2r, b2r = b1.reshape(1, H), W2.reshape(1, H), b2.reshape(1, 1)

    outs = []
    for s in range(NSPLIT):
        flat = _sc_gather(tab, fi[s])
        emb3 = flat.reshape(CT, BS, 128)
        outs.append(_mlp(emb3, num_p[s * BS:(s + 1) * BS], mean_p, std_p,
                         w1e, w1n, b1r, w2r, b2r))
    return jnp.concatenate(outs, axis=0)


# speedup vs baseline: 2.6221x; 1.0813x over previous
import functools

import jax
import jax.numpy as jnp
from jax import lax
from jax.experimental import pallas as pl
from jax.experimental.pallas import tpu as pltpu
from jax.experimental.pallas import tpu_sc as plsc

B = 16384
F = 26
V = 1000
D = 64
NUM = 13
H = 1024
FD = F * D
CT = FD // 128
NUMP = 128

NSPLIT = 4
BS = B // NSPLIT
BFS = BS * F

NC, NS = 2, 16
NW = NC * NS
CHUNK = 128
NCHUNKS = BFS // CHUNK
CH = NCHUNKS // NW

_sc_mesh = plsc.VectorSubcoreMesh(core_axis_name="c", subcore_axis_name="s")


@functools.partial(
    pl.kernel,
    out_type=jax.ShapeDtypeStruct((BFS, D), jnp.float32),
    mesh=_sc_mesh,
    scratch_types=[
        pltpu.VMEM((CH, CHUNK), jnp.int32),
        pltpu.VMEM((4, CHUNK, D), jnp.float32),
        pltpu.SemaphoreType.DMA,
        pltpu.SemaphoreType.DMA,
    ],
    compiler_params=pltpu.CompilerParams(use_tc_tiling_on_sc=False),
)
def _sc_gather(table_hbm, idx_hbm, out_hbm, idx_v, rows_v, gsem, osem):
    wid = lax.axis_index("s") * NC + lax.axis_index("c")
    pltpu.sync_copy(idx_hbm.at[pl.ds(wid * CH, CH)], idx_v)
    base_row = wid * CH * CHUNK

    def out_slice(j):
        return out_hbm.at[pl.ds(base_row + j * CHUNK, CHUNK)]

    def gather(j, buf):
        return pltpu.async_copy(table_hbm.at[idx_v.at[j]], rows_v.at[buf], gsem)

    gather(0, 0)
    gather(1, 1)

    def body(j, carry):
        buf = lax.rem(j, 4)
        pltpu.make_async_copy(table_hbm.at[idx_v.at[j]], rows_v.at[buf],
                              gsem).wait()
        pltpu.async_copy(rows_v.at[buf], out_slice(j), osem)
        @pl.when(j >= 2)
        def _():
            pltpu.make_async_copy(rows_v.at[lax.rem(j - 2, 4)],
                                  out_slice(j - 2), osem).wait()

        @pl.when(j + 2 < CH)
        def _():
            gather(j + 2, lax.rem(j + 2, 4))
        return carry

    lax.fori_loop(0, CH, body, 0)
    pltpu.make_async_copy(rows_v.at[(CH - 2) % 4], out_slice(CH - 2), osem).wait()
    pltpu.make_async_copy(rows_v.at[(CH - 1) % 4], out_slice(CH - 1), osem).wait()


def _mlp_body(emb_ref, num_ref, mean_ref, std_ref, w1e_ref, w1n_ref,
              b1_ref, w2_ref, b2_ref, out_ref):
    num = (num_ref[...] - mean_ref[...]) / std_ref[...]
    acc = jnp.dot(num, w1n_ref[...], preferred_element_type=jnp.float32)
    emb = jnp.concatenate(
        [emb_ref[c].astype(jnp.bfloat16) for c in range(CT)], axis=1)
    acc = acc + jnp.dot(emb, w1e_ref[...], preferred_element_type=jnp.float32)
    x = jnp.maximum(acc + b1_ref[...], 0.0)
    out_ref[...] = jnp.sum(x * w2_ref[...], axis=1, keepdims=True) + b2_ref[...]


BB = 512


def _mlp(emb3, num_p, mean_p, std_p, w1e, w1n, b1r, w2r, b2r):
    grid = (BS // BB,)
    return pl.pallas_call(
        _mlp_body,
        grid=grid,
        in_specs=[
            pl.BlockSpec((CT, BB, 128), lambda i: (0, i, 0)),
            pl.BlockSpec((BB, NUMP), lambda i: (i, 0)),
            pl.BlockSpec((1, NUMP), lambda i: (0, 0)),
            pl.BlockSpec((1, NUMP), lambda i: (0, 0)),
            pl.BlockSpec((FD, H), lambda i: (0, 0)),
            pl.BlockSpec((NUMP, H), lambda i: (0, 0)),
            pl.BlockSpec((1, H), lambda i: (0, 0)),
            pl.BlockSpec((1, H), lambda i: (0, 0)),
            pl.BlockSpec((1, 1), lambda i: (0, 0)),
        ],
        out_specs=pl.BlockSpec((BB, 1), lambda i: (i, 0)),
        out_shape=jax.ShapeDtypeStruct((BS, 1), jnp.float32),
    )(emb3, num_p, mean_p, std_p, w1e, w1n, b1r, w2r, b2r)


def kernel(cat_indices, numericals, emb_tables, norm_mean, norm_std, W1, b1, W2, b2):
    tab = emb_tables.reshape(F * V, D)
    offs = (jnp.arange(F, dtype=jnp.int32) * V)[None, :]
    fi = cat_indices.astype(jnp.int32) + offs
    fi = fi.reshape(NSPLIT, BS // 64, 64, CT, 2).transpose(0, 3, 1, 2, 4)
    fi = fi.reshape(NSPLIT, NCHUNKS, CHUNK)

    num_p = jnp.pad(numericals, ((0, 0), (0, NUMP - NUM)))
    mean_p = jnp.pad(norm_mean, (0, NUMP - NUM)).reshape(1, NUMP)
    std_p = jnp.pad(norm_std, (0, NUMP - NUM), constant_values=1.0).reshape(1, NUMP)
    w1e = W1[:FD].astype(jnp.bfloat16)
    w1n = jnp.pad(W1[FD:], ((0, NUMP - NUM), (0, 0)))
    b1r, w2r, b2r = b1.reshape(1, H), W2.reshape(1, H), b2.reshape(1, 1)

    outs = []
    for s in range(NSPLIT):
        flat = _sc_gather(tab, fi[s])
        emb3 = flat.reshape(CT, BS, 128)
        outs.append(_mlp(emb3, num_p[s * BS:(s + 1) * BS], mean_p, std_p,
                         w1e, w1n, b1r, w2r, b2r))
    return jnp.concatenate(outs, axis=0)
